# baseline (speedup 1.0000x reference)
import functools

import jax
import jax.numpy as jnp
import numpy as np
from jax import lax
from jax.experimental import pallas as pl
from jax.experimental.pallas import tpu as pltpu
from jax.experimental.pallas import tpu_sc as plsc

N_NODES = 10000
N_EDGES = 320000
D = 128
DH = D // 2

NC = 2
NS = 16
L = 16

CH = 96
NCH = 210
E_PAD = NS * NCH * CH
N_PAD = 10240
ROWS_PER_SUB = N_PAD // NS

_PERM = np.concatenate([
    np.arange(0, 32, 2), np.arange(1, 32, 2),
    np.arange(32, 64, 2), np.arange(33, 64, 2),
])

_mesh = plsc.VectorSubcoreMesh(core_axis_name="c", subcore_axis_name="s")


@functools.partial(
    pl.kernel,
    out_type=jax.ShapeDtypeStruct((NC, N_PAD, DH), jnp.float32),
    mesh=_mesh,
    compiler_params=pltpu.CompilerParams(use_tc_tiling_on_sc=False, needs_layout_passes=False),
    scratch_types=[
        pltpu.VMEM((NCH, CH), jnp.int32),
        pltpu.VMEM((NCH, CH), jnp.int32),
        pltpu.VMEM((NCH + 1, CH), jnp.float32),
        pltpu.VMEM((CH, DH), jnp.bfloat16),
        pltpu.VMEM((CH, DH), jnp.bfloat16),
        pltpu.VMEM((CH, DH), jnp.bfloat16),
        pltpu.VMEM((CH, DH), jnp.float32),
        pltpu.VMEM((CH, DH), jnp.float32),
        pltpu.VMEM((CH, DH), jnp.float32),
        pltpu.VMEM_SHARED((N_PAD, DH), jnp.float32),
        pltpu.SemaphoreType.DMA,
        pltpu.SemaphoreType.DMA,
        pltpu.SemaphoreType.DMA,
        pltpu.SemaphoreType.DMA,
        pltpu.SemaphoreType.DMA,
        pltpu.SemaphoreType.DMA,
        pltpu.SemaphoreType.DMA,
        pltpu.SemaphoreType.DMA,
        pltpu.SemaphoreType.DMA,
    ],
)
def _sc_aggregate(x2_hbm, col_hbm, row_hbm, w_hbm, zeros_hbm, out_hbm,
                  col_s, row_s, w_s, bf0, bf1, bf2, sc0, sc1, sc2, acc_sh,
                  g0, g1, g2, h0, h1, h2, s0, s1, s2):
    c = lax.axis_index("c")
    s = lax.axis_index("s")
    bfs = (bf0, bf1, bf2)
    scs = (sc0, sc1, sc2)
    gsem = (g0, g1, g2)
    hsem = (h0, h1, h2)
    ssem = (s0, s1, s2)
    x_half = x2_hbm.at[c]

    pltpu.sync_copy(
        zeros_hbm,
        acc_sh.at[pl.ds(s * ROWS_PER_SUB, ROWS_PER_SUB)],
    )
    pltpu.sync_copy(col_hbm.at[s], col_s)
    pltpu.sync_copy(row_hbm.at[s], row_s)
    pltpu.sync_copy(w_hbm.at[s], w_s.at[pl.ds(0, NCH)])
    plsc.subcore_barrier()

    HC = CH // 2

    def issue_gather(k, b):
        pltpu.async_copy(x_half.at[col_s.at[k, pl.ds(0, HC)]],
                         bfs[b].at[pl.ds(0, HC)], gsem[b])
        pltpu.async_copy(x_half.at[col_s.at[k, pl.ds(HC, HC)]],
                         bfs[b].at[pl.ds(HC, HC)], hsem[b])

    def issue_scatter(k, b):
        pltpu.async_copy(scs[b], acc_sh.at[row_s.at[k]], ssem[b], add=True)

    def wait_gather(b):
        pltpu.make_async_copy(x_half.at[col_s.at[0, pl.ds(0, HC)]],
                              bfs[b].at[pl.ds(0, HC)], gsem[b]).wait()
        pltpu.make_async_copy(x_half.at[col_s.at[0, pl.ds(HC, HC)]],
                              bfs[b].at[pl.ds(HC, HC)], hsem[b]).wait()

    def wait_scatter(b):
        pltpu.make_async_copy(scs[b], acc_sh.at[row_s.at[0]], ssem[b]).wait()

    def scale(k, b):
        bv = bfs[b]
        sv = scs[b]

        @plsc.parallel_loop(0, CH, step=1, unroll=4)
        def _scale_edge(e):
            wj = w_s[k, pl.ds(e, L)][0]
            for h in range(DH // 32):
                packed = bv[e, pl.ds(h * 32, 32)]
                lo, hi = plsc.unpack(packed, format=plsc.PackFormat.INTERLEAVED,
                                     preferred_element_type=jnp.float32)
                sv[e, pl.ds(h * 32, L)] = lo * wj
                sv[e, pl.ds(h * 32 + L, L)] = hi * wj

    issue_gather(0, 0)
    issue_gather(1, 1)

    def chunk_body(t, carry):
        for b in range(3):
            k = t * 3 + b
            wait_gather(b)
            if b == 0:
                issue_gather(k + 2, 2)
            else:
                @pl.when(t < NCH // 3 - 1)
                def _():
                    issue_gather(k + 2, (b + 2) % 3)
            scale(k, b)
            if b == 0:
                @pl.when(t >= 1)
                def _():
                    wait_scatter(2)
            else:
                wait_scatter(b - 1)
            issue_scatter(k, b)
        return carry

    lax.fori_loop(0, NCH // 3, chunk_body, 0)

    wait_scatter((NCH - 1) % 3)

    plsc.subcore_barrier()
    pltpu.sync_copy(
        acc_sh.at[pl.ds(s * ROWS_PER_SUB, ROWS_PER_SUB)],
        out_hbm.at[c, pl.ds(s * ROWS_PER_SUB, ROWS_PER_SUB)],
    )


ROWS_BLK = 1000


def _tc_finish(p_ref, w_ref, b_ref, o_ref):
    y = jnp.dot(p_ref[0], w_ref[0], preferred_element_type=jnp.float32)
    y = y + jnp.dot(p_ref[1], w_ref[1], preferred_element_type=jnp.float32)
    o_ref[...] = jnp.maximum(y + b_ref[...], 0.0)


def kernel(x, edge_index, edge_weight, kernel, bias):
    row = edge_index[0].astype(jnp.int32)
    col = edge_index[1].astype(jnp.int32)
    x2 = jnp.stack([x[:, :DH], x[:, DH:]]).astype(jnp.bfloat16)
    pad = E_PAD - N_EDGES
    col_p = jnp.pad(col, (0, pad)).reshape(NS, NCH, CH)
    row_p = jnp.pad(row, (0, pad)).reshape(NS, NCH, CH)
    w_p = jnp.pad(edge_weight.astype(jnp.float32), (0, pad)).reshape(
        NS, NCH, CH)
    zeros = jnp.zeros((ROWS_PER_SUB, DH), jnp.float32)
    partials = _sc_aggregate(x2, col_p, row_p, w_p, zeros)
    w2 = jnp.stack([kernel[:DH][_PERM], kernel[DH:][_PERM]])
    out = pl.pallas_call(
        _tc_finish,
        grid=(N_NODES // ROWS_BLK,),
        in_specs=[
            pl.BlockSpec((NC, ROWS_BLK, DH), lambda i: (0, i, 0)),
            pl.BlockSpec((NC, DH, D), lambda i: (0, 0, 0)),
            pl.BlockSpec((1, D), lambda i: (0, 0)),
        ],
        out_specs=pl.BlockSpec((ROWS_BLK, D), lambda i: (i, 0)),
        out_shape=jax.ShapeDtypeStruct((N_NODES, D), jnp.float32),
    )(partials, w2, bias.reshape(1, D))
    return out


# speedup vs baseline: 1.0022x; 1.0022x over previous
import functools

import jax
import jax.numpy as jnp
import numpy as np
from jax import lax
from jax.experimental import pallas as pl
from jax.experimental.pallas import tpu as pltpu
from jax.experimental.pallas import tpu_sc as plsc

N_NODES = 10000
N_EDGES = 320000
D = 128
DH = D // 2

NC = 2
NS = 16
L = 16

CH = 96
NCH = 210
E_PAD = NS * NCH * CH
N_PAD = 10240
ROWS_PER_SUB = N_PAD // NS

_PERM = np.concatenate([
    np.arange(0, 32, 2), np.arange(1, 32, 2),
    np.arange(32, 64, 2), np.arange(33, 64, 2),
])

_mesh = plsc.VectorSubcoreMesh(core_axis_name="c", subcore_axis_name="s")


@functools.partial(
    pl.kernel,
    out_type=jax.ShapeDtypeStruct((NC, N_PAD, DH), jnp.float32),
    mesh=_mesh,
    compiler_params=pltpu.CompilerParams(use_tc_tiling_on_sc=False, needs_layout_passes=False),
    scratch_types=[
        pltpu.VMEM((NCH, CH), jnp.int32),
        pltpu.VMEM((NCH, CH), jnp.int32),
        pltpu.VMEM((NCH + 1, CH), jnp.float32),
        pltpu.VMEM((CH, DH), jnp.bfloat16),
        pltpu.VMEM((CH, DH), jnp.bfloat16),
        pltpu.VMEM((CH, DH), jnp.bfloat16),
        pltpu.VMEM((CH, DH), jnp.float32),
        pltpu.VMEM((CH, DH), jnp.float32),
        pltpu.VMEM((CH, DH), jnp.float32),
        pltpu.VMEM_SHARED((N_PAD, DH), jnp.float32),
        pltpu.SemaphoreType.DMA,
        pltpu.SemaphoreType.DMA,
        pltpu.SemaphoreType.DMA,
        pltpu.SemaphoreType.DMA,
        pltpu.SemaphoreType.DMA,
        pltpu.SemaphoreType.DMA,
    ],
)
def _sc_aggregate(x2_hbm, col_hbm, row_hbm, w_hbm, zeros_hbm, out_hbm,
                  col_s, row_s, w_s, bf0, bf1, bf2, sc0, sc1, sc2, acc_sh,
                  g0, g1, g2, s0, s1, s2):
    c = lax.axis_index("c")
    s = lax.axis_index("s")
    bfs = (bf0, bf1, bf2)
    scs = (sc0, sc1, sc2)
    gsem = (g0, g1, g2)
    ssem = (s0, s1, s2)
    x_half = x2_hbm.at[c]

    pltpu.sync_copy(
        zeros_hbm,
        acc_sh.at[pl.ds(s * ROWS_PER_SUB, ROWS_PER_SUB)],
    )
    pltpu.sync_copy(col_hbm.at[s], col_s)
    pltpu.sync_copy(row_hbm.at[s], row_s)
    pltpu.sync_copy(w_hbm.at[s], w_s.at[pl.ds(0, NCH)])
    plsc.subcore_barrier()

    def issue_gather(k, b):
        pltpu.async_copy(x_half.at[col_s.at[k]], bfs[b], gsem[b])

    def issue_scatter(k, b):
        pltpu.async_copy(scs[b], acc_sh.at[row_s.at[k]], ssem[b], add=True)

    def wait_gather(b):
        pltpu.make_async_copy(x_half.at[col_s.at[0]], bfs[b], gsem[b]).wait()

    def wait_scatter(b):
        pltpu.make_async_copy(scs[b], acc_sh.at[row_s.at[0]], ssem[b]).wait()

    def scale(k, b):
        bv = bfs[b]
        sv = scs[b]

        @plsc.parallel_loop(0, CH, step=1, unroll=4)
        def _scale_edge(e):
            wj = w_s[k, pl.ds(e, L)][0]
            for h in range(DH // 32):
                packed = bv[e, pl.ds(h * 32, 32)]
                lo, hi = plsc.unpack(packed, format=plsc.PackFormat.INTERLEAVED,
                                     preferred_element_type=jnp.float32)
                sv[e, pl.ds(h * 32, L)] = lo * wj
                sv[e, pl.ds(h * 32 + L, L)] = hi * wj

    issue_gather(0, 0)
    issue_gather(1, 1)

    def chunk_body(t, carry):
        for b in range(3):
            k = t * 3 + b
            wait_gather(b)
            if b == 0:
                issue_gather(k + 2, 2)
            else:
                @pl.when(t < NCH // 3 - 1)
                def _():
                    issue_gather(k + 2, (b + 2) % 3)
            scale(k, b)
            if b == 0:
                @pl.when(t >= 1)
                def _():
                    wait_scatter(2)
            else:
                wait_scatter(b - 1)
            issue_scatter(k, b)
        return carry

    lax.fori_loop(0, NCH // 3, chunk_body, 0)

    wait_scatter((NCH - 1) % 3)

    plsc.subcore_barrier()
    pltpu.sync_copy(
        acc_sh.at[pl.ds(s * ROWS_PER_SUB, ROWS_PER_SUB)],
        out_hbm.at[c, pl.ds(s * ROWS_PER_SUB, ROWS_PER_SUB)],
    )


ROWS_BLK = 1000


def _tc_finish(p_ref, w_ref, b_ref, o_ref):
    y = jnp.dot(p_ref[0], w_ref[0], preferred_element_type=jnp.float32)
    y = y + jnp.dot(p_ref[1], w_ref[1], preferred_element_type=jnp.float32)
    o_ref[...] = jnp.maximum(y + b_ref[...], 0.0)


def kernel(x, edge_index, edge_weight, kernel, bias):
    row = edge_index[0].astype(jnp.int32)
    col = edge_index[1].astype(jnp.int32)
    x2 = jnp.stack([x[:, :DH], x[:, DH:]]).astype(jnp.bfloat16)
    pad = E_PAD - N_EDGES
    col_p = jnp.pad(col, (0, pad)).reshape(NS, NCH, CH)
    row_p = jnp.pad(row, (0, pad)).reshape(NS, NCH, CH)
    w_p = jnp.pad(edge_weight.astype(jnp.float32), (0, pad)).reshape(
        NS, NCH, CH)
    zeros = jnp.zeros((ROWS_PER_SUB, DH), jnp.float32)
    partials = _sc_aggregate(x2, col_p, row_p, w_p, zeros)
    w2 = jnp.stack([kernel[:DH][_PERM], kernel[DH:][_PERM]])
    out = pl.pallas_call(
        _tc_finish,
        grid=(N_NODES // ROWS_BLK,),
        in_specs=[
            pl.BlockSpec((NC, ROWS_BLK, DH), lambda i: (0, i, 0)),
            pl.BlockSpec((NC, DH, D), lambda i: (0, 0, 0)),
            pl.BlockSpec((1, D), lambda i: (0, 0)),
        ],
        out_specs=pl.BlockSpec((ROWS_BLK, D), lambda i: (i, 0)),
        out_shape=jax.ShapeDtypeStruct((N_NODES, D), jnp.float32),
    )(partials, w2, bias.reshape(1, D))
    return out


# speedup vs baseline: 1.0681x; 1.0658x over previous
import functools

import jax
import jax.numpy as jnp
import numpy as np
from jax import lax
from jax.experimental import pallas as pl
from jax.experimental.pallas import tpu as pltpu
from jax.experimental.pallas import tpu_sc as plsc

N_NODES = 10000
N_EDGES = 320000
D = 128
DH = D // 2

NC = 2
NS = 16
L = 16

CH = 96
NCH = 210
E_PAD = NS * NCH * CH
N_PAD = 10240
ROWS_PER_SUB = N_PAD // NS

_PERM = np.concatenate([
    np.arange(0, 32, 2), np.arange(1, 32, 2),
    np.arange(32, 64, 2), np.arange(33, 64, 2),
])

_mesh = plsc.VectorSubcoreMesh(core_axis_name="c", subcore_axis_name="s")


@functools.partial(
    pl.kernel,
    out_type=jax.ShapeDtypeStruct((NC, N_PAD, DH), jnp.float32),
    mesh=_mesh,
    compiler_params=pltpu.CompilerParams(use_tc_tiling_on_sc=False, needs_layout_passes=False),
    scratch_types=[
        pltpu.VMEM((NCH, CH), jnp.int32),
        pltpu.VMEM((NCH, CH), jnp.int32),
        pltpu.VMEM((NCH + 1, CH), jnp.float32),
        pltpu.VMEM((CH, DH), jnp.bfloat16),
        pltpu.VMEM((CH, DH), jnp.bfloat16),
        pltpu.VMEM((CH, DH), jnp.bfloat16),
        pltpu.VMEM((CH, DH), jnp.float32),
        pltpu.VMEM((CH, DH), jnp.float32),
        pltpu.VMEM((CH, DH), jnp.float32),
        pltpu.VMEM_SHARED((N_PAD, DH), jnp.float32),
        pltpu.SemaphoreType.DMA,
        pltpu.SemaphoreType.DMA,
        pltpu.SemaphoreType.DMA,
        pltpu.SemaphoreType.DMA,
        pltpu.SemaphoreType.DMA,
        pltpu.SemaphoreType.DMA,
    ],
)
def _sc_aggregate(x2_hbm, ei_hbm, w_hbm, zeros_hbm, out_hbm,
                  col_s, row_s, w_s, bf0, bf1, bf2, sc0, sc1, sc2, acc_sh,
                  g0, g1, g2, s0, s1, s2):
    c = lax.axis_index("c")
    s = lax.axis_index("s")
    bfs = (bf0, bf1, bf2)
    scs = (sc0, sc1, sc2)
    gsem = (g0, g1, g2)
    ssem = (s0, s1, s2)
    x_half = x2_hbm.at[c]

    pltpu.sync_copy(
        zeros_hbm,
        acc_sh.at[pl.ds(s * ROWS_PER_SUB, ROWS_PER_SUB)],
    )
    pltpu.sync_copy(ei_hbm.at[1, s], col_s)
    pltpu.sync_copy(ei_hbm.at[0, s], row_s)
    pltpu.sync_copy(w_hbm.at[s], w_s.at[pl.ds(0, NCH)])
    plsc.subcore_barrier()

    def issue_gather(k, b):
        pltpu.async_copy(x_half.at[col_s.at[k]], bfs[b], gsem[b])

    def issue_scatter(k, b):
        pltpu.async_copy(scs[b], acc_sh.at[row_s.at[k]], ssem[b], add=True)

    def wait_gather(b):
        pltpu.make_async_copy(x_half.at[col_s.at[0]], bfs[b], gsem[b]).wait()

    def wait_scatter(b):
        pltpu.make_async_copy(scs[b], acc_sh.at[row_s.at[0]], ssem[b]).wait()

    def scale(k, b):
        bv = bfs[b]
        sv = scs[b]

        @plsc.parallel_loop(0, CH, step=1, unroll=4)
        def _scale_edge(e):
            wj = w_s[k, pl.ds(e, L)][0]
            for h in range(DH // 32):
                packed = bv[e, pl.ds(h * 32, 32)]
                lo, hi = plsc.unpack(packed, format=plsc.PackFormat.INTERLEAVED,
                                     preferred_element_type=jnp.float32)
                sv[e, pl.ds(h * 32, L)] = lo * wj
                sv[e, pl.ds(h * 32 + L, L)] = hi * wj

    issue_gather(0, 0)
    issue_gather(1, 1)

    def chunk_body(t, carry):
        for b in range(3):
            k = t * 3 + b
            wait_gather(b)
            if b == 0:
                issue_gather(k + 2, 2)
            else:
                @pl.when(t < NCH // 3 - 1)
                def _():
                    issue_gather(k + 2, (b + 2) % 3)
            scale(k, b)
            if b == 0:
                @pl.when(t >= 1)
                def _():
                    wait_scatter(2)
            else:
                wait_scatter(b - 1)
            issue_scatter(k, b)
        return carry

    lax.fori_loop(0, NCH // 3, chunk_body, 0)

    wait_scatter((NCH - 1) % 3)

    plsc.subcore_barrier()
    pltpu.sync_copy(
        acc_sh.at[pl.ds(s * ROWS_PER_SUB, ROWS_PER_SUB)],
        out_hbm.at[c, pl.ds(s * ROWS_PER_SUB, ROWS_PER_SUB)],
    )


ROWS_BLK = 1000


def _tc_finish(p_ref, w_ref, b_ref, o_ref):
    y = jnp.dot(p_ref[0], w_ref[0], preferred_element_type=jnp.float32)
    y = y + jnp.dot(p_ref[1], w_ref[1], preferred_element_type=jnp.float32)
    o_ref[...] = jnp.maximum(y + b_ref[...], 0.0)


def kernel(x, edge_index, edge_weight, kernel, bias):
    x2 = jnp.stack([x[:, :DH], x[:, DH:]]).astype(jnp.bfloat16)
    pad = E_PAD - N_EDGES
    ei_p = jnp.pad(edge_index.astype(jnp.int32),
                   ((0, 0), (0, pad))).reshape(2, NS, NCH, CH)
    w_p = jnp.pad(edge_weight.astype(jnp.float32), (0, pad)).reshape(
        NS, NCH, CH)
    zeros = jnp.zeros((ROWS_PER_SUB, DH), jnp.float32)
    partials = _sc_aggregate(x2, ei_p, w_p, zeros)
    w2 = jnp.stack([kernel[:DH][_PERM], kernel[DH:][_PERM]])
    out = pl.pallas_call(
        _tc_finish,
        grid=(N_NODES // ROWS_BLK,),
        in_specs=[
            pl.BlockSpec((NC, ROWS_BLK, DH), lambda i: (0, i, 0)),
            pl.BlockSpec((NC, DH, D), lambda i: (0, 0, 0)),
            pl.BlockSpec((1, D), lambda i: (0, 0)),
        ],
        out_specs=pl.BlockSpec((ROWS_BLK, D), lambda i: (i, 0)),
        out_shape=jax.ShapeDtypeStruct((N_NODES, D), jnp.float32),
    )(partials, w2, bias.reshape(1, D))
    return out
